# baseline (device time: 9099 ns/iter reference)
import jax
import jax.numpy as jnp
from jax import lax
from jax.experimental import pallas as pl
from jax.experimental.pallas import tpu as pltpu

N_DEV = 4
N_TOK = 256
D_IN = 128
D_OUT = 256
CAP = 25
BLK = N_TOK // N_DEV


def kernel(x, router_W, route_idx, expert_W):
    del router_W

    def body(x_ref, idx_ref, w_ref, out_ref,
             partial_ref, recv_ref, send_sems, recv_sems):
        my = lax.axis_index("i")

        bsem = pltpu.get_barrier_semaphore()
        for d in range(1, N_DEV):
            pl.semaphore_signal(
                bsem, inc=1,
                device_id=((my + d) % N_DEV,),
                device_id_type=pl.DeviceIdType.MESH,
            )
        pl.semaphore_wait(bsem, N_DEV - 1)

        r = idx_ref[:, :]
        row = lax.broadcasted_iota(jnp.int32, (N_TOK, N_TOK), 0)
        col = lax.broadcasted_iota(jnp.int32, (N_TOK, N_TOK), 1)
        tri = (col <= row).astype(jnp.float32)

        e0 = 2 * my
        m0 = (r == e0).astype(jnp.float32)
        m1 = (r == e0 + 1).astype(jnp.float32)
        cum0 = jnp.dot(tri, m0, preferred_element_type=jnp.float32)
        cum1 = jnp.dot(tri, m1, preferred_element_type=jnp.float32)
        a0 = m0 * (cum0 <= float(CAP)).astype(jnp.float32)
        a1 = m1 * (cum1 <= float(CAP)).astype(jnp.float32)

        xv = x_ref[:, :]
        p0 = jnp.dot(xv, w_ref[0], preferred_element_type=jnp.float32)
        p1 = jnp.dot(xv, w_ref[1], preferred_element_type=jnp.float32)
        partial_ref[:, :] = a0 * p0 + a1 * p1

        rdmas = []
        for d in range(1, N_DEV):
            tgt = (my + d) % N_DEV
            rdma = pltpu.make_async_remote_copy(
                src_ref=partial_ref.at[pl.ds(tgt * BLK, BLK), :],
                dst_ref=recv_ref.at[d - 1],
                send_sem=send_sems.at[d - 1],
                recv_sem=recv_sems.at[d - 1],
                device_id=(tgt,),
                device_id_type=pl.DeviceIdType.MESH,
            )
            rdma.start()
            rdmas.append(rdma)

        acc = partial_ref[pl.ds(my * BLK, BLK), :]
        for d in range(1, N_DEV):
            rdmas[d - 1].wait_recv()
            acc = acc + recv_ref[d - 1]
        out_ref[:, :] = acc

        for d in range(1, N_DEV):
            rdmas[d - 1].wait_send()

    return pl.pallas_call(
        body,
        out_shape=jax.ShapeDtypeStruct((BLK, D_OUT), jnp.float32),
        in_specs=[
            pl.BlockSpec(memory_space=pltpu.VMEM),
            pl.BlockSpec(memory_space=pltpu.VMEM),
            pl.BlockSpec(memory_space=pltpu.VMEM),
        ],
        out_specs=pl.BlockSpec(memory_space=pltpu.VMEM),
        scratch_shapes=[
            pltpu.VMEM((N_TOK, D_OUT), jnp.float32),
            pltpu.VMEM((N_DEV - 1, BLK, D_OUT), jnp.float32),
            pltpu.SemaphoreType.DMA((N_DEV - 1,)),
            pltpu.SemaphoreType.DMA((N_DEV - 1,)),
        ],
        compiler_params=pltpu.CompilerParams(collective_id=0),
    )(x, route_idx, expert_W)


# device time: 9040 ns/iter; 1.0065x vs baseline; 1.0065x over previous
import jax
import jax.numpy as jnp
from jax import lax
from jax.experimental import pallas as pl
from jax.experimental.pallas import tpu as pltpu

N_DEV = 4
N_TOK = 256
D_IN = 128
D_OUT = 256
CAP = 25
BLK = N_TOK // N_DEV


def kernel(x, router_W, route_idx, expert_W):
    del router_W

    def body(x_ref, idx_ref, w_ref, out_ref,
             partial_ref, mask_ref, recv_ref, send_sems, recv_sems):
        my = lax.axis_index("i")

        bsem = pltpu.get_barrier_semaphore()
        for d in range(1, N_DEV):
            pl.semaphore_signal(
                bsem, inc=1,
                device_id=((my + d) % N_DEV,),
                device_id_type=pl.DeviceIdType.MESH,
            )

        r = idx_ref[:, :]
        row = lax.broadcasted_iota(jnp.int32, (N_TOK, N_TOK), 0)
        col = lax.broadcasted_iota(jnp.int32, (N_TOK, N_TOK), 1)
        tri = (col <= row).astype(jnp.float32)

        e0 = 2 * my
        m = jnp.concatenate(
            [(r == e0).astype(jnp.float32),
             (r == e0 + 1).astype(jnp.float32)], axis=1)
        cum = jnp.dot(tri, m, preferred_element_type=jnp.float32)
        mask_ref[:, :] = m * (cum <= float(CAP)).astype(jnp.float32)

        pl.semaphore_wait(bsem, N_DEV - 1)

        w0 = w_ref[0]
        w1 = w_ref[1]

        def block_out(start):
            xb = x_ref[pl.ds(start, BLK), :]
            a0 = mask_ref[pl.ds(start, BLK), 0:1]
            a1 = mask_ref[pl.ds(start, BLK), 1:2]
            return (jnp.dot(a0 * xb, w0, preferred_element_type=jnp.float32)
                    + jnp.dot(a1 * xb, w1, preferred_element_type=jnp.float32))

        rdmas = {}
        for d in (2, 1, 3):
            tgt = (my + d) % N_DEV
            start = tgt * BLK
            partial_ref[pl.ds(start, BLK), :] = block_out(start)
            rdma = pltpu.make_async_remote_copy(
                src_ref=partial_ref.at[pl.ds(start, BLK), :],
                dst_ref=recv_ref.at[d - 1],
                send_sem=send_sems.at[d - 1],
                recv_sem=recv_sems.at[d - 1],
                device_id=(tgt,),
                device_id_type=pl.DeviceIdType.MESH,
            )
            rdma.start()
            rdmas[d] = rdma

        acc = block_out(my * BLK)
        for d in range(1, N_DEV):
            rdmas[d].wait_recv()
            acc = acc + recv_ref[d - 1]
        out_ref[:, :] = acc

        for d in range(1, N_DEV):
            rdmas[d].wait_send()

    return pl.pallas_call(
        body,
        out_shape=jax.ShapeDtypeStruct((BLK, D_OUT), jnp.float32),
        in_specs=[
            pl.BlockSpec(memory_space=pltpu.VMEM),
            pl.BlockSpec(memory_space=pltpu.VMEM),
            pl.BlockSpec(memory_space=pltpu.VMEM),
        ],
        out_specs=pl.BlockSpec(memory_space=pltpu.VMEM),
        scratch_shapes=[
            pltpu.VMEM((N_TOK, D_OUT), jnp.float32),
            pltpu.VMEM((N_TOK, 2), jnp.float32),
            pltpu.VMEM((N_DEV - 1, BLK, D_OUT), jnp.float32),
            pltpu.SemaphoreType.DMA((N_DEV - 1,)),
            pltpu.SemaphoreType.DMA((N_DEV - 1,)),
        ],
        compiler_params=pltpu.CompilerParams(collective_id=0),
    )(x, route_idx, expert_W)
